# baseline (device time: 90557 ns/iter reference)
import jax
import jax.numpy as jnp
from jax import lax
from jax.experimental import pallas as pl
from jax.experimental.pallas import tpu as pltpu

N_DEV = 8


def kernel(x, w_mat):
    m, k_shard = x.shape
    _, n = w_mat.shape
    m_chunk = m // N_DEV

    def body(
        x_ref, w_ref, out_ref, q_recv_h,
        q_send, q_recv, staging, s_send, s_recv,
        q_send_sems, q_recv_sems, s_send_sems, s_recv_sems, cp_sems,
    ):
        my = lax.axis_index("i")
        w = w_ref[:, :]

        rdmas = []
        for k in range(1, N_DEV):
            dst = (my + k) % N_DEV
            row0 = dst * m_chunk
            part = jnp.dot(
                x_ref[pl.ds(row0, m_chunk), :], w,
                preferred_element_type=jnp.float32,
            )
            absmax = jnp.maximum(jnp.max(jnp.abs(part)), 1e-30)
            q_send[k - 1, :, :] = jnp.clip(
                jnp.round(part * (127.0 / absmax)), -127.0, 127.0
            ).astype(jnp.int8)
            s_send[k - 1, :, :] = (absmax * (1.0 / 127.0)) * jnp.ones(
                (1, 128), jnp.float32
            )
            dst_space = q_recv if k % 2 == 1 else q_recv_h
            q_rdma = pltpu.make_async_remote_copy(
                src_ref=q_send.at[k - 1],
                dst_ref=dst_space.at[k - 1],
                send_sem=q_send_sems.at[k - 1],
                recv_sem=q_recv_sems.at[k - 1],
                device_id=(dst,),
                device_id_type=pl.DeviceIdType.MESH,
            )
            s_rdma = pltpu.make_async_remote_copy(
                src_ref=s_send.at[k - 1],
                dst_ref=s_recv.at[k - 1],
                send_sem=s_send_sems.at[k - 1],
                recv_sem=s_recv_sems.at[k - 1],
                device_id=(dst,),
                device_id_type=pl.DeviceIdType.MESH,
            )
            q_rdma.start()
            s_rdma.start()
            rdmas.append((q_rdma, s_rdma))

        out_ref[:, :] = jnp.dot(
            x_ref[pl.ds(my * m_chunk, m_chunk), :], w,
            preferred_element_type=jnp.float32,
        )

        for k in (1, 3, 5, 7):
            j = k - 1
            rdmas[j][0].wait_recv()
            rdmas[j][1].wait_recv()
            out_ref[:, :] = out_ref[:, :] + (
                q_recv[j, :, :].astype(jnp.float32) * s_recv[j, 0, 0]
            )

        hbm_js = [1, 3, 5]
        copies = []
        for slot, j in enumerate(hbm_js):
            rdmas[j][0].wait_recv()
            cp = pltpu.make_async_copy(
                q_recv_h.at[j], staging.at[slot], cp_sems.at[slot]
            )
            cp.start()
            copies.append(cp)
        for slot, j in enumerate(hbm_js):
            copies[slot].wait()
            rdmas[j][1].wait_recv()
            out_ref[:, :] = out_ref[:, :] + (
                staging[slot, :, :].astype(jnp.float32) * s_recv[j, 0, 0]
            )

        out_ref[:, :] = jnp.maximum(out_ref[:, :], 0.0)

        for j in range(N_DEV - 1):
            rdmas[j][0].wait_send()
            rdmas[j][1].wait_send()

    out, _ = pl.pallas_call(
        body,
        out_shape=(
            jax.ShapeDtypeStruct((m_chunk, n), jnp.float32),
            jax.ShapeDtypeStruct((N_DEV - 1, m_chunk, n), jnp.int8),
        ),
        in_specs=[
            pl.BlockSpec(memory_space=pltpu.VMEM),
            pl.BlockSpec(memory_space=pltpu.VMEM),
        ],
        out_specs=(
            pl.BlockSpec(memory_space=pltpu.VMEM),
            pl.BlockSpec(memory_space=pltpu.HBM),
        ),
        scratch_shapes=[
            pltpu.VMEM((N_DEV - 1, m_chunk, n), jnp.int8),
            pltpu.VMEM((N_DEV - 1, m_chunk, n), jnp.int8),
            pltpu.VMEM((3, m_chunk, n), jnp.int8),
            pltpu.VMEM((N_DEV - 1, 1, 128), jnp.float32),
            pltpu.VMEM((N_DEV - 1, 1, 128), jnp.float32),
            pltpu.SemaphoreType.DMA((N_DEV - 1,)),
            pltpu.SemaphoreType.DMA((N_DEV - 1,)),
            pltpu.SemaphoreType.DMA((N_DEV - 1,)),
            pltpu.SemaphoreType.DMA((N_DEV - 1,)),
            pltpu.SemaphoreType.DMA((3,)),
        ],
    )(x, w_mat)
    return out


# device time: 85667 ns/iter; 1.0571x vs baseline; 1.0571x over previous
import jax
import jax.numpy as jnp
from jax import lax
from jax.experimental import pallas as pl
from jax.experimental.pallas import tpu as pltpu

N_DEV = 8


def kernel(x, w_mat):
    m, k_shard = x.shape
    _, n = w_mat.shape
    m_chunk = m // N_DEV

    def body(
        x_ref, w_ref, out_ref,
        q_send, q_recv, s_send, s_recv,
        q_send_sems, q_recv_sems, s_send_sems, s_recv_sems,
    ):
        my = lax.axis_index("i")
        w = w_ref[:, :]

        rdmas = []
        for k in range(1, N_DEV):
            dst = (my + k) % N_DEV
            row0 = dst * m_chunk
            part = jnp.dot(
                x_ref[pl.ds(row0, m_chunk), :], w,
                preferred_element_type=jnp.float32,
            )
            absmax = jnp.maximum(jnp.max(jnp.abs(part)), 1e-30)
            q_send[k - 1, :, :] = jnp.clip(
                jnp.round(part * (127.0 / absmax)), -127.0, 127.0
            ).astype(jnp.int8)
            s_send[k - 1, :, :] = (absmax * (1.0 / 127.0)) * jnp.ones(
                (1, 128), jnp.float32
            )
            q_rdma = pltpu.make_async_remote_copy(
                src_ref=q_send.at[k - 1],
                dst_ref=q_recv.at[k - 1],
                send_sem=q_send_sems.at[k - 1],
                recv_sem=q_recv_sems.at[k - 1],
                device_id=(dst,),
                device_id_type=pl.DeviceIdType.MESH,
            )
            s_rdma = pltpu.make_async_remote_copy(
                src_ref=s_send.at[k - 1],
                dst_ref=s_recv.at[k - 1],
                send_sem=s_send_sems.at[k - 1],
                recv_sem=s_recv_sems.at[k - 1],
                device_id=(dst,),
                device_id_type=pl.DeviceIdType.MESH,
            )
            q_rdma.start()
            s_rdma.start()
            rdmas.append((q_rdma, s_rdma))

        out_ref[:, :] = jnp.dot(
            x_ref[pl.ds(my * m_chunk, m_chunk), :], w,
            preferred_element_type=jnp.float32,
        )

        for j in range(N_DEV - 1):
            q_rdma, s_rdma = rdmas[j]
            q_rdma.wait_recv()
            s_rdma.wait_recv()
            acc = out_ref[:, :] + (
                q_recv[j, :, :].astype(jnp.float32) * s_recv[j, 0, 0]
            )
            if j == N_DEV - 2:
                acc = jnp.maximum(acc, 0.0)
            out_ref[:, :] = acc

        for j in range(N_DEV - 1):
            rdmas[j][0].wait_send()
            rdmas[j][1].wait_send()

    return pl.pallas_call(
        body,
        out_shape=jax.ShapeDtypeStruct((m_chunk, n), jnp.float32),
        in_specs=[
            pl.BlockSpec(memory_space=pltpu.VMEM),
            pl.BlockSpec(memory_space=pltpu.VMEM),
        ],
        out_specs=pl.BlockSpec(memory_space=pltpu.VMEM),
        scratch_shapes=[
            pltpu.VMEM((N_DEV - 1, m_chunk, n), jnp.int8),
            pltpu.VMEM((N_DEV - 1, m_chunk, n), jnp.int8),
            pltpu.VMEM((N_DEV - 1, 1, 128), jnp.float32),
            pltpu.VMEM((N_DEV - 1, 1, 128), jnp.float32),
            pltpu.SemaphoreType.DMA((N_DEV - 1,)),
            pltpu.SemaphoreType.DMA((N_DEV - 1,)),
            pltpu.SemaphoreType.DMA((N_DEV - 1,)),
            pltpu.SemaphoreType.DMA((N_DEV - 1,)),
        ],
    )(x, w_mat)


# device time: 83569 ns/iter; 1.0836x vs baseline; 1.0251x over previous
import jax
import jax.numpy as jnp
from jax import lax
from jax.experimental import pallas as pl
from jax.experimental.pallas import tpu as pltpu

N_DEV = 8


def kernel(x, w_mat):
    m, k_shard = x.shape
    _, n = w_mat.shape
    m_chunk = m // N_DEV
    m_half = m_chunk // 2

    def body(
        x_ref, w_ref, out_ref,
        acc, q_send, q_recv, s_send, s_recv,
        q_send_sems, q_recv_sems, s_send_sems, s_recv_sems,
    ):
        my = lax.axis_index("i")
        w = w_ref[:, :]

        def quant_and_send(slot, sem, row0, rows, scale_slot):
            part = jnp.dot(
                x_ref[pl.ds(((my + slot + 1) % N_DEV) * m_chunk + row0, rows), :],
                w,
                preferred_element_type=jnp.float32,
            )
            absmax = jnp.maximum(jnp.max(jnp.abs(part)), 1e-30)
            q_send[slot, pl.ds(row0, rows), :] = jnp.clip(
                jnp.round(part * (127.0 / absmax)), -127.0, 127.0
            ).astype(jnp.int8)
            s_send[scale_slot, :, :] = (absmax * (1.0 / 127.0)) * jnp.ones(
                (1, 128), jnp.float32
            )
            dst = (my + slot + 1) % N_DEV
            q_rdma = pltpu.make_async_remote_copy(
                src_ref=q_send.at[slot, pl.ds(row0, rows)],
                dst_ref=q_recv.at[slot, pl.ds(row0, rows)],
                send_sem=q_send_sems.at[sem],
                recv_sem=q_recv_sems.at[sem],
                device_id=(dst,),
                device_id_type=pl.DeviceIdType.MESH,
            )
            s_rdma = pltpu.make_async_remote_copy(
                src_ref=s_send.at[scale_slot],
                dst_ref=s_recv.at[scale_slot],
                send_sem=s_send_sems.at[scale_slot],
                recv_sem=s_recv_sems.at[scale_slot],
                device_id=(dst,),
                device_id_type=pl.DeviceIdType.MESH,
            )
            q_rdma.start()
            s_rdma.start()
            return (q_rdma, s_rdma)

        rdmas = []
        first_a = quant_and_send(0, 0, 0, m_half, 0)
        first_b = quant_and_send(0, N_DEV - 1, m_half, m_half, N_DEV - 1)
        rdmas.append((first_a, first_b))
        for k in range(2, N_DEV):
            rdmas.append(quant_and_send(k - 1, k - 1, 0, m_chunk, k - 1))

        acc[:, :] = jnp.dot(
            x_ref[pl.ds(my * m_chunk, m_chunk), :], w,
            preferred_element_type=jnp.float32,
        )

        (qa, sa), (qb, sb) = rdmas[0]
        qa.wait_recv()
        sa.wait_recv()
        acc[pl.ds(0, m_half), :] = acc[pl.ds(0, m_half), :] + (
            q_recv[0, pl.ds(0, m_half), :].astype(jnp.float32) * s_recv[0, 0, 0]
        )
        qb.wait_recv()
        sb.wait_recv()
        acc[pl.ds(m_half, m_half), :] = acc[pl.ds(m_half, m_half), :] + (
            q_recv[0, pl.ds(m_half, m_half), :].astype(jnp.float32)
            * s_recv[N_DEV - 1, 0, 0]
        )
        for j in range(1, N_DEV - 1):
            q_rdma, s_rdma = rdmas[j]
            q_rdma.wait_recv()
            s_rdma.wait_recv()
            acc[:, :] = acc[:, :] + (
                q_recv[j, :, :].astype(jnp.float32) * s_recv[j, 0, 0]
            )

        out_ref[:, :] = jnp.maximum(acc[:, :], 0.0).astype(jnp.bfloat16)

        qa.wait_send()
        sa.wait_send()
        qb.wait_send()
        sb.wait_send()
        for j in range(1, N_DEV - 1):
            rdmas[j][0].wait_send()
            rdmas[j][1].wait_send()

    return pl.pallas_call(
        body,
        out_shape=jax.ShapeDtypeStruct((m_chunk, n), jnp.bfloat16),
        in_specs=[
            pl.BlockSpec(memory_space=pltpu.VMEM),
            pl.BlockSpec(memory_space=pltpu.VMEM),
        ],
        out_specs=pl.BlockSpec(memory_space=pltpu.VMEM),
        scratch_shapes=[
            pltpu.VMEM((m_chunk, n), jnp.float32),
            pltpu.VMEM((N_DEV - 1, m_chunk, n), jnp.int8),
            pltpu.VMEM((N_DEV - 1, m_chunk, n), jnp.int8),
            pltpu.VMEM((N_DEV, 1, 128), jnp.float32),
            pltpu.VMEM((N_DEV, 1, 128), jnp.float32),
            pltpu.SemaphoreType.DMA((N_DEV,)),
            pltpu.SemaphoreType.DMA((N_DEV,)),
            pltpu.SemaphoreType.DMA((N_DEV,)),
            pltpu.SemaphoreType.DMA((N_DEV,)),
        ],
    )(x, w_mat)
